# baseline (speedup 1.0000x reference)
import functools

import jax
import jax.numpy as jnp
import numpy as np
from jax import lax
from jax.experimental import pallas as pl
from jax.experimental.pallas import tpu as pltpu
from jax.experimental.pallas import tpu_sc as plsc

N = 10000
NPAD = 10240
D = 128
E = 320000

NC = 2
NS = 16
NW = NC * NS
CHUNK = 128
NCHUNK = 80
EPW = NCHUNK * CHUNK
EPAD = NW * EPW
SUPER = 8
SUPN = NCHUNK // SUPER
ROWS_PER_TILE = NPAD // NS

_MESH = plsc.VectorSubcoreMesh(core_axis_name="c", subcore_axis_name="s")


@functools.partial(
    pl.kernel,
    out_type=(
        jax.ShapeDtypeStruct((NC, NPAD, D), jnp.float32),
        jax.ShapeDtypeStruct((NW * NPAD // D, D), jnp.float32),
    ),
    mesh=_MESH,
    compiler_params=pltpu.CompilerParams(use_tc_tiling_on_sc=True,
                                         needs_layout_passes=False),
    scratch_types=[
        pltpu.VMEM((SUPER, CHUNK), jnp.int32),
        pltpu.VMEM((SUPER, CHUNK), jnp.int32),
        pltpu.VMEM((SUPER, CHUNK), jnp.int32),
        pltpu.VMEM((SUPER, CHUNK), jnp.int32),
        pltpu.VMEM((CHUNK, D), jnp.float32),
        pltpu.VMEM((CHUNK, D), jnp.float32),
        pltpu.VMEM((NPAD // D, D), jnp.float32),
        pltpu.VMEM_SHARED((NPAD, D), jnp.float32),
        pltpu.SemaphoreType.DMA,
        pltpu.SemaphoreType.DMA,
        pltpu.SemaphoreType.DMA,
        pltpu.SemaphoreType.DMA,
    ],
)
def _sc_aggregate(table_hbm, src_hbm, dst_hbm, zeros_hbm,
                  feat_hbm, deg_hbm,
                  src_a, dst_a, src_b, dst_b, rows0_v, rows1_v, deg_v,
                  acc_sh, sem0, sem1, sem_is, sem_id):
    c = lax.axis_index("c")
    s = lax.axis_index("s")
    w = c * NS + s
    r0 = s * ROWS_PER_TILE
    base = w * NCHUNK
    pltpu.sync_copy(zeros_hbm, acc_sh.at[pl.ds(r0, ROWS_PER_TILE)])
    pltpu.sync_copy(zeros_hbm.at[pl.ds(0, NPAD // D)], deg_v)
    pltpu.sync_copy(src_hbm.at[pl.ds(base, SUPER)], src_a)
    pltpu.sync_copy(dst_hbm.at[pl.ds(base, SUPER)], dst_a)
    plsc.subcore_barrier()

    ones16 = jnp.ones((16,), jnp.float32)

    def process_super(u, src_c, dst_c, src_n, dst_n):
        @pl.when(u + 1 < SUPN)
        def _():
            off = base + (u + 1) * SUPER
            pltpu.async_copy(src_hbm.at[pl.ds(off, SUPER)], src_n, sem_is)
            pltpu.async_copy(dst_hbm.at[pl.ds(off, SUPER)], dst_n, sem_id)

        pltpu.async_copy(table_hbm.at[src_c.at[0]], rows0_v, sem0)

        def dbody(k, carry):
            idx = dst_c[k // 8, pl.ds((k % 8) * 16, 16)]
            plsc.addupdate_scatter(deg_v, [idx >> 7, idx & 127], ones16)
            return carry

        lax.fori_loop(0, SUPER * 8, dbody, 0)

        def inner(i, carry):
            j0 = 2 * i
            pltpu.async_copy(table_hbm.at[src_c.at[j0 + 1]], rows1_v, sem1)
            pltpu.make_async_copy(table_hbm.at[src_c.at[j0]], rows0_v,
                                  sem0).wait()
            pltpu.sync_copy(rows0_v, acc_sh.at[dst_c.at[j0]], add=True)

            @pl.when(j0 + 2 < SUPER)
            def _():
                pltpu.async_copy(table_hbm.at[src_c.at[j0 + 2]], rows0_v,
                                 sem0)

            pltpu.make_async_copy(table_hbm.at[src_c.at[j0 + 1]], rows1_v,
                                  sem1).wait()
            pltpu.sync_copy(rows1_v, acc_sh.at[dst_c.at[j0 + 1]], add=True)
            return carry

        lax.fori_loop(0, SUPER // 2, inner, 0)

        @pl.when(u + 1 < SUPN)
        def _():
            pltpu.make_async_copy(src_hbm.at[pl.ds(base, SUPER)], src_n,
                                  sem_is).wait()
            pltpu.make_async_copy(dst_hbm.at[pl.ds(base, SUPER)], dst_n,
                                  sem_id).wait()

    def super_body(t, carry):
        u0 = 2 * t
        process_super(u0, src_a, dst_a, src_b, dst_b)
        process_super(u0 + 1, src_b, dst_b, src_a, dst_a)
        return carry

    lax.fori_loop(0, SUPN // 2, super_body, 0)
    plsc.subcore_barrier()
    pltpu.sync_copy(acc_sh.at[pl.ds(r0, ROWS_PER_TILE)],
                    feat_hbm.at[c, pl.ds(r0, ROWS_PER_TILE)])
    pltpu.sync_copy(deg_v, deg_hbm.at[pl.ds(w * (NPAD // D), NPAD // D)])


_RBLK = 1024


def _dense_body(relu, h_ref, p_ref, degp_ref, ws_ref, wn_ref, b_ref, o_ref):
    h = h_ref[...]
    feat = p_ref[0] + p_ref[1]
    deg = lax.dot_general(degp_ref[...], jnp.ones((NW, 1), jnp.float32),
                          (((0,), (0,)), ((), ())),
                          preferred_element_type=jnp.float32)
    hn = feat / jnp.maximum(deg, 1.0)
    act = (jnp.dot(h, ws_ref[...], preferred_element_type=jnp.float32)
           + jnp.dot(hn, wn_ref[...], preferred_element_type=jnp.float32)
           + b_ref[...])
    if relu:
        act = jnp.maximum(act, 0.0)
    o_ref[...] = act


def _dense(h, p, degp, w_self, w_neigh, b, relu):
    grid = (NPAD // _RBLK,)
    return pl.pallas_call(
        functools.partial(_dense_body, relu),
        grid=grid,
        in_specs=[
            pl.BlockSpec((_RBLK, D), lambda i: (i, 0)),
            pl.BlockSpec((NC, _RBLK, D), lambda i: (0, i, 0)),
            pl.BlockSpec((NW, _RBLK), lambda i: (0, i)),
            pl.BlockSpec((D, D), lambda i: (0, 0)),
            pl.BlockSpec((D, D), lambda i: (0, 0)),
            pl.BlockSpec((1, D), lambda i: (0, 0)),
        ],
        out_specs=pl.BlockSpec((_RBLK, D), lambda i: (i, 0)),
        out_shape=jax.ShapeDtypeStruct((NPAD, D), jnp.float32),
    )(h, p, degp, w_self, w_neigh, b)


_PAD = EPAD - E
_SRC_TMPL = np.zeros((EPAD,), np.int32)
_SRC_TMPL[E:] = np.arange(_PAD, dtype=np.int32) % N
_DST_TMPL = np.zeros((EPAD,), np.int32)
_DST_TMPL[E:] = N + np.arange(_PAD, dtype=np.int32) % (NPAD - N)


def _pad_edges(ei):
    src = lax.dynamic_update_slice(jnp.asarray(_SRC_TMPL),
                                   ei[0].astype(jnp.int32), (0,))
    dst = lax.dynamic_update_slice(jnp.asarray(_DST_TMPL),
                                   ei[1].astype(jnp.int32), (0,))
    return src.reshape(-1, CHUNK), dst.reshape(-1, CHUNK)


def kernel(x, edge_index0, edge_index1, W_self1, W_neigh1, b1,
           W_self2, W_neigh2, b2):
    src0, dst0 = _pad_edges(edge_index0)
    src1, dst1 = _pad_edges(edge_index1)
    zeros = jnp.zeros((ROWS_PER_TILE, D), jnp.float32)
    xp = jnp.pad(x, ((0, NPAD - N), (0, 0)))
    b1r = b1.reshape(1, D)
    b2r = b2.reshape(1, D)

    p1, d1 = _sc_aggregate(x, src0, dst0, zeros)
    h = _dense(xp, p1, d1.reshape(NW, NPAD), W_self1, W_neigh1, b1r,
               relu=True)
    p2, d2 = _sc_aggregate(h, src1, dst1, zeros)
    out = _dense(h, p2, d2.reshape(NW, NPAD), W_self2, W_neigh2, b2r,
                 relu=False)
    return out[:N]


# speedup vs baseline: 1.0294x; 1.0294x over previous
import functools

import jax
import jax.numpy as jnp
import numpy as np
from jax import lax
from jax.experimental import pallas as pl
from jax.experimental.pallas import tpu as pltpu
from jax.experimental.pallas import tpu_sc as plsc

N = 10000
NPAD = 10240
D = 128
E = 320000

NC = 2
NS = 16
NW = NC * NS
CHUNK = 128
NCHUNK = 80
EPW = NCHUNK * CHUNK
EPAD = NW * EPW
SUPER = 8
SUPN = NCHUNK // SUPER
ROWS_PER_TILE = NPAD // NS

_MESH = plsc.VectorSubcoreMesh(core_axis_name="c", subcore_axis_name="s")


@functools.partial(
    pl.kernel,
    out_type=(
        jax.ShapeDtypeStruct((NC, NPAD, D), jnp.float32),
        jax.ShapeDtypeStruct((NW * NPAD // D, D), jnp.float32),
    ),
    mesh=_MESH,
    compiler_params=pltpu.CompilerParams(use_tc_tiling_on_sc=True,
                                         needs_layout_passes=False),
    scratch_types=[
        pltpu.VMEM((SUPER, CHUNK), jnp.int32),
        pltpu.VMEM((SUPER, CHUNK), jnp.int32),
        pltpu.VMEM((SUPER, CHUNK), jnp.int32),
        pltpu.VMEM((SUPER, CHUNK), jnp.int32),
        pltpu.VMEM((CHUNK, D), jnp.float32),
        pltpu.VMEM((CHUNK, D), jnp.float32),
        pltpu.VMEM((NPAD // D, D), jnp.float32),
        pltpu.VMEM_SHARED((NPAD, D), jnp.float32),
        pltpu.SemaphoreType.DMA,
        pltpu.SemaphoreType.DMA,
        pltpu.SemaphoreType.DMA,
        pltpu.SemaphoreType.DMA,
    ],
)
def _sc_aggregate(table_hbm, src_hbm, dst_hbm, zeros_hbm,
                  feat_hbm, deg_hbm,
                  src_a, dst_a, src_b, dst_b, rows0_v, rows1_v, deg_v,
                  acc_sh, sem0, sem1, sem_is, sem_id):
    c = lax.axis_index("c")
    s = lax.axis_index("s")
    w = c * NS + s
    r0 = s * ROWS_PER_TILE
    base = w * NCHUNK
    pltpu.sync_copy(zeros_hbm.at[pl.ds(r0, ROWS_PER_TILE)],
                    acc_sh.at[pl.ds(r0, ROWS_PER_TILE)])
    pltpu.sync_copy(zeros_hbm.at[pl.ds(r0, NPAD // D)], deg_v)
    pltpu.sync_copy(src_hbm.at[pl.ds(base, SUPER)], src_a)
    pltpu.sync_copy(dst_hbm.at[pl.ds(base, SUPER)], dst_a)
    plsc.subcore_barrier()

    ones16 = jnp.ones((16,), jnp.float32)

    def process_super(u, src_c, dst_c, src_n, dst_n):
        @pl.when(u + 1 < SUPN)
        def _():
            off = base + (u + 1) * SUPER
            pltpu.async_copy(src_hbm.at[pl.ds(off, SUPER)], src_n, sem_is)
            pltpu.async_copy(dst_hbm.at[pl.ds(off, SUPER)], dst_n, sem_id)

        pltpu.async_copy(table_hbm.at[src_c.at[0]], rows0_v, sem0)

        def dbody(k, carry):
            idx = dst_c[k // 8, pl.ds((k % 8) * 16, 16)]
            plsc.addupdate_scatter(deg_v, [idx >> 7, idx & 127], ones16)
            return carry

        lax.fori_loop(0, SUPER * 8, dbody, 0)

        def inner(i, carry):
            j0 = 2 * i
            pltpu.async_copy(table_hbm.at[src_c.at[j0 + 1]], rows1_v, sem1)
            pltpu.make_async_copy(table_hbm.at[src_c.at[j0]], rows0_v,
                                  sem0).wait()
            pltpu.sync_copy(rows0_v, acc_sh.at[dst_c.at[j0]], add=True)

            @pl.when(j0 + 2 < SUPER)
            def _():
                pltpu.async_copy(table_hbm.at[src_c.at[j0 + 2]], rows0_v,
                                 sem0)

            pltpu.make_async_copy(table_hbm.at[src_c.at[j0 + 1]], rows1_v,
                                  sem1).wait()
            pltpu.sync_copy(rows1_v, acc_sh.at[dst_c.at[j0 + 1]], add=True)
            return carry

        lax.fori_loop(0, SUPER // 2, inner, 0)

        @pl.when(u + 1 < SUPN)
        def _():
            pltpu.make_async_copy(src_hbm.at[pl.ds(base, SUPER)], src_n,
                                  sem_is).wait()
            pltpu.make_async_copy(dst_hbm.at[pl.ds(base, SUPER)], dst_n,
                                  sem_id).wait()

    def super_body(t, carry):
        u0 = 2 * t
        process_super(u0, src_a, dst_a, src_b, dst_b)
        process_super(u0 + 1, src_b, dst_b, src_a, dst_a)
        return carry

    lax.fori_loop(0, SUPN // 2, super_body, 0)
    plsc.subcore_barrier()
    pltpu.sync_copy(acc_sh.at[pl.ds(r0, ROWS_PER_TILE)],
                    feat_hbm.at[c, pl.ds(r0, ROWS_PER_TILE)])
    pltpu.sync_copy(deg_v, deg_hbm.at[pl.ds(w * (NPAD // D), NPAD // D)])


_RBLK = 1024


def _dense_body(relu, h_ref, p_ref, degp_ref, ws_ref, wn_ref, b_ref, o_ref):
    h = h_ref[...]
    feat = p_ref[0] + p_ref[1]
    deg = lax.dot_general(degp_ref[...], jnp.ones((NW, 1), jnp.float32),
                          (((0,), (0,)), ((), ())),
                          preferred_element_type=jnp.float32)
    hn = feat / jnp.maximum(deg, 1.0)
    act = (jnp.dot(h, ws_ref[...], preferred_element_type=jnp.float32)
           + jnp.dot(hn, wn_ref[...], preferred_element_type=jnp.float32)
           + b_ref[...])
    if relu:
        act = jnp.maximum(act, 0.0)
    o_ref[...] = act


def _dense(h, p, degp, w_self, w_neigh, b, relu):
    grid = (NPAD // _RBLK,)
    return pl.pallas_call(
        functools.partial(_dense_body, relu),
        grid=grid,
        in_specs=[
            pl.BlockSpec((_RBLK, D), lambda i: (i, 0)),
            pl.BlockSpec((NC, _RBLK, D), lambda i: (0, i, 0)),
            pl.BlockSpec((NW, _RBLK), lambda i: (0, i)),
            pl.BlockSpec((D, D), lambda i: (0, 0)),
            pl.BlockSpec((D, D), lambda i: (0, 0)),
            pl.BlockSpec((1, D), lambda i: (0, 0)),
        ],
        out_specs=pl.BlockSpec((_RBLK, D), lambda i: (i, 0)),
        out_shape=jax.ShapeDtypeStruct((NPAD, D), jnp.float32),
    )(h, p, degp, w_self, w_neigh, b)


_PAD = EPAD - E
_SRC_TMPL = np.zeros((EPAD,), np.int32)
_SRC_TMPL[E:] = np.arange(_PAD, dtype=np.int32) % N
_DST_TMPL = np.zeros((EPAD,), np.int32)
_DST_TMPL[E:] = N + np.arange(_PAD, dtype=np.int32) % (NPAD - N)


def _pad_edges(ei):
    src = lax.dynamic_update_slice(jnp.asarray(_SRC_TMPL),
                                   ei[0].astype(jnp.int32), (0,))
    dst = lax.dynamic_update_slice(jnp.asarray(_DST_TMPL),
                                   ei[1].astype(jnp.int32), (0,))
    return src.reshape(-1, CHUNK), dst.reshape(-1, CHUNK)


def kernel(x, edge_index0, edge_index1, W_self1, W_neigh1, b1,
           W_self2, W_neigh2, b2):
    src0, dst0 = _pad_edges(edge_index0)
    src1, dst1 = _pad_edges(edge_index1)
    zeros = jnp.zeros((NPAD, D), jnp.float32)
    xp = jnp.pad(x, ((0, NPAD - N), (0, 0)))
    b1r = b1.reshape(1, D)
    b2r = b2.reshape(1, D)

    p1, d1 = _sc_aggregate(x, src0, dst0, zeros)
    h = _dense(xp, p1, d1.reshape(NW, NPAD), W_self1, W_neigh1, b1r,
               relu=True)
    p2, d2 = _sc_aggregate(h, src1, dst1, zeros)
    out = _dense(h, p2, d2.reshape(NW, NPAD), W_self2, W_neigh2, b2r,
                 relu=False)
    return out[:N]


# speedup vs baseline: 1.0756x; 1.0449x over previous
import functools

import jax
import jax.numpy as jnp
import numpy as np
from jax import lax
from jax.experimental import pallas as pl
from jax.experimental.pallas import tpu as pltpu
from jax.experimental.pallas import tpu_sc as plsc

N = 10000
NPAD = 10240
D = 128
E = 320000

NC = 2
NS = 16
NW = NC * NS
CHUNK = 128
NCHUNK = 80
EPW = NCHUNK * CHUNK
EPAD = NW * EPW
SUPER = 8
SUPN = NCHUNK // SUPER
ROWS_PER_TILE = NPAD // NS

_MESH = plsc.VectorSubcoreMesh(core_axis_name="c", subcore_axis_name="s")


@functools.partial(
    pl.kernel,
    out_type=(
        jax.ShapeDtypeStruct((NC, NPAD, D), jnp.float32),
        jax.ShapeDtypeStruct((NW * NPAD // D, D), jnp.float32),
    ),
    mesh=_MESH,
    compiler_params=pltpu.CompilerParams(use_tc_tiling_on_sc=True,
                                         needs_layout_passes=False),
    scratch_types=[
        pltpu.VMEM((SUPER, CHUNK), jnp.int32),
        pltpu.VMEM((SUPER, CHUNK), jnp.int32),
        pltpu.VMEM((SUPER, CHUNK), jnp.int32),
        pltpu.VMEM((SUPER, CHUNK), jnp.int32),
        pltpu.VMEM((CHUNK, D), jnp.float32),
        pltpu.VMEM((CHUNK, D), jnp.float32),
        pltpu.VMEM((NPAD // D, D), jnp.float32),
        pltpu.VMEM_SHARED((NPAD, D), jnp.float32),
        pltpu.SemaphoreType.DMA,
        pltpu.SemaphoreType.DMA,
        pltpu.SemaphoreType.DMA,
        pltpu.SemaphoreType.DMA,
    ],
)
def _sc_aggregate(table_hbm, src_hbm, dst_hbm, zeros_hbm,
                  feat_hbm, deg_hbm,
                  src_a, dst_a, src_b, dst_b, rows0_v, rows1_v, deg_v,
                  acc_sh, sem0, sem1, sem_is, sem_id):
    c = lax.axis_index("c")
    s = lax.axis_index("s")
    w = c * NS + s
    r0 = s * ROWS_PER_TILE
    base = w * NCHUNK
    pltpu.sync_copy(zeros_hbm.at[pl.ds(r0, ROWS_PER_TILE)],
                    acc_sh.at[pl.ds(r0, ROWS_PER_TILE)])
    pltpu.sync_copy(zeros_hbm.at[pl.ds(r0, NPAD // D)], deg_v)
    pltpu.sync_copy(src_hbm.at[pl.ds(base, SUPER)], src_a)
    pltpu.sync_copy(dst_hbm.at[pl.ds(base, SUPER)], dst_a)
    plsc.subcore_barrier()

    ones16 = jnp.ones((16,), jnp.float32)

    pltpu.async_copy(table_hbm.at[src_a.at[0]], rows0_v, sem0)

    def process_super(u, src_c, dst_c, src_n, dst_n):
        @pl.when(u + 1 < SUPN)
        def _():
            off = base + (u + 1) * SUPER
            pltpu.async_copy(src_hbm.at[pl.ds(off, SUPER)], src_n, sem_is)
            pltpu.async_copy(dst_hbm.at[pl.ds(off, SUPER)], dst_n, sem_id)

        def dbody(k, carry):
            idx = dst_c[k // 8, pl.ds((k % 8) * 16, 16)]
            plsc.addupdate_scatter(deg_v, [idx >> 7, idx & 127], ones16)
            return carry

        lax.fori_loop(0, SUPER * 8, dbody, 0)

        def inner(i, carry):
            j0 = 2 * i
            pltpu.async_copy(table_hbm.at[src_c.at[j0 + 1]], rows1_v, sem1)
            pltpu.make_async_copy(table_hbm.at[src_c.at[j0]], rows0_v,
                                  sem0).wait()
            pltpu.sync_copy(rows0_v, acc_sh.at[dst_c.at[j0]], add=True)

            @pl.when(j0 + 2 < SUPER)
            def _():
                pltpu.async_copy(table_hbm.at[src_c.at[j0 + 2]], rows0_v,
                                 sem0)

            pltpu.make_async_copy(table_hbm.at[src_c.at[j0 + 1]], rows1_v,
                                  sem1).wait()
            pltpu.sync_copy(rows1_v, acc_sh.at[dst_c.at[j0 + 1]], add=True)
            return carry

        lax.fori_loop(0, SUPER // 2 - 1, inner, 0)

        jl = SUPER - 2
        pltpu.async_copy(table_hbm.at[src_c.at[jl + 1]], rows1_v, sem1)
        pltpu.make_async_copy(table_hbm.at[src_c.at[jl]], rows0_v,
                              sem0).wait()
        pltpu.sync_copy(rows0_v, acc_sh.at[dst_c.at[jl]], add=True)

        @pl.when(u + 1 < SUPN)
        def _():
            pltpu.make_async_copy(src_hbm.at[pl.ds(base, SUPER)], src_n,
                                  sem_is).wait()
            pltpu.make_async_copy(dst_hbm.at[pl.ds(base, SUPER)], dst_n,
                                  sem_id).wait()
            pltpu.async_copy(table_hbm.at[src_n.at[0]], rows0_v, sem0)

        pltpu.make_async_copy(table_hbm.at[src_c.at[jl + 1]], rows1_v,
                              sem1).wait()
        pltpu.sync_copy(rows1_v, acc_sh.at[dst_c.at[jl + 1]], add=True)

    def super_body(t, carry):
        u0 = 2 * t
        process_super(u0, src_a, dst_a, src_b, dst_b)
        process_super(u0 + 1, src_b, dst_b, src_a, dst_a)
        return carry

    lax.fori_loop(0, SUPN // 2, super_body, 0)
    plsc.subcore_barrier()
    pltpu.sync_copy(acc_sh.at[pl.ds(r0, ROWS_PER_TILE)],
                    feat_hbm.at[c, pl.ds(r0, ROWS_PER_TILE)])
    pltpu.sync_copy(deg_v, deg_hbm.at[pl.ds(w * (NPAD // D), NPAD // D)])


_RBLK = 1024


def _dense_body(relu, h_ref, p_ref, degp_ref, ws_ref, wn_ref, b_ref, o_ref):
    h = h_ref[...]
    feat = p_ref[0] + p_ref[1]
    deg = lax.dot_general(degp_ref[...], jnp.ones((NW, 1), jnp.float32),
                          (((0,), (0,)), ((), ())),
                          preferred_element_type=jnp.float32)
    hn = feat / jnp.maximum(deg, 1.0)
    act = (jnp.dot(h, ws_ref[...], preferred_element_type=jnp.float32)
           + jnp.dot(hn, wn_ref[...], preferred_element_type=jnp.float32)
           + b_ref[...])
    if relu:
        act = jnp.maximum(act, 0.0)
    o_ref[...] = act


def _dense(h, p, degp, w_self, w_neigh, b, relu):
    grid = (NPAD // _RBLK,)
    return pl.pallas_call(
        functools.partial(_dense_body, relu),
        grid=grid,
        in_specs=[
            pl.BlockSpec((_RBLK, D), lambda i: (i, 0)),
            pl.BlockSpec((NC, _RBLK, D), lambda i: (0, i, 0)),
            pl.BlockSpec((NW, _RBLK), lambda i: (0, i)),
            pl.BlockSpec((D, D), lambda i: (0, 0)),
            pl.BlockSpec((D, D), lambda i: (0, 0)),
            pl.BlockSpec((1, D), lambda i: (0, 0)),
        ],
        out_specs=pl.BlockSpec((_RBLK, D), lambda i: (i, 0)),
        out_shape=jax.ShapeDtypeStruct((NPAD, D), jnp.float32),
    )(h, p, degp, w_self, w_neigh, b)


_PAD = EPAD - E
_SRC_TMPL = np.zeros((EPAD,), np.int32)
_SRC_TMPL[E:] = np.arange(_PAD, dtype=np.int32) % N
_DST_TMPL = np.zeros((EPAD,), np.int32)
_DST_TMPL[E:] = N + np.arange(_PAD, dtype=np.int32) % (NPAD - N)


def _pad_edges(ei):
    src = lax.dynamic_update_slice(jnp.asarray(_SRC_TMPL),
                                   ei[0].astype(jnp.int32), (0,))
    dst = lax.dynamic_update_slice(jnp.asarray(_DST_TMPL),
                                   ei[1].astype(jnp.int32), (0,))
    return src.reshape(-1, CHUNK), dst.reshape(-1, CHUNK)


def kernel(x, edge_index0, edge_index1, W_self1, W_neigh1, b1,
           W_self2, W_neigh2, b2):
    src0, dst0 = _pad_edges(edge_index0)
    src1, dst1 = _pad_edges(edge_index1)
    zeros = jnp.zeros((NPAD, D), jnp.float32)
    xp = jnp.pad(x, ((0, NPAD - N), (0, 0)))
    b1r = b1.reshape(1, D)
    b2r = b2.reshape(1, D)

    p1, d1 = _sc_aggregate(x, src0, dst0, zeros)
    h = _dense(xp, p1, d1.reshape(NW, NPAD), W_self1, W_neigh1, b1r,
               relu=True)
    p2, d2 = _sc_aggregate(h, src1, dst1, zeros)
    out = _dense(h, p2, d2.reshape(NW, NPAD), W_self2, W_neigh2, b2r,
                 relu=False)
    return out[:N]


# speedup vs baseline: 1.0764x; 1.0008x over previous
import functools

import jax
import jax.numpy as jnp
import numpy as np
from jax import lax
from jax.experimental import pallas as pl
from jax.experimental.pallas import tpu as pltpu
from jax.experimental.pallas import tpu_sc as plsc

N = 10000
NPAD = 10240
D = 128
E = 320000

NC = 2
NS = 16
NW = NC * NS
CHUNK = 128
NCHUNK = 80
EPW = NCHUNK * CHUNK
EPAD = NW * EPW
SUPER = 8
SUPN = NCHUNK // SUPER
ROWS_PER_TILE = NPAD // NS

_MESH = plsc.VectorSubcoreMesh(core_axis_name="c", subcore_axis_name="s")


@functools.partial(
    pl.kernel,
    out_type=(
        jax.ShapeDtypeStruct((NC, NPAD, D), jnp.float32),
        jax.ShapeDtypeStruct((NW * NPAD // D, D), jnp.float32),
    ),
    mesh=_MESH,
    compiler_params=pltpu.CompilerParams(use_tc_tiling_on_sc=True,
                                         needs_layout_passes=False),
    scratch_types=[
        pltpu.VMEM((SUPER, CHUNK), jnp.int32),
        pltpu.VMEM((SUPER, CHUNK), jnp.int32),
        pltpu.VMEM((SUPER, CHUNK), jnp.int32),
        pltpu.VMEM((SUPER, CHUNK), jnp.int32),
        pltpu.VMEM((CHUNK, D), jnp.float32),
        pltpu.VMEM((CHUNK, D), jnp.float32),
        pltpu.VMEM((NPAD // D, D), jnp.float32),
        pltpu.VMEM_SHARED((NPAD, D), jnp.float32),
        pltpu.SemaphoreType.DMA,
        pltpu.SemaphoreType.DMA,
        pltpu.SemaphoreType.DMA,
        pltpu.SemaphoreType.DMA,
    ],
)
def _sc_aggregate(table_hbm, src_hbm, dst_hbm, zeros_hbm,
                  feat_hbm, deg_hbm,
                  src_a, dst_a, src_b, dst_b, rows0_v, rows1_v, deg_v,
                  acc_sh, sem0, sem1, sem_is, sem_id):
    c = lax.axis_index("c")
    s = lax.axis_index("s")
    w = c * NS + s
    r0 = s * ROWS_PER_TILE
    base = w * NCHUNK
    pltpu.sync_copy(zeros_hbm.at[pl.ds(r0, ROWS_PER_TILE)],
                    acc_sh.at[pl.ds(r0, ROWS_PER_TILE)])
    pltpu.sync_copy(zeros_hbm.at[pl.ds(r0, NPAD // D)], deg_v)
    pltpu.sync_copy(src_hbm.at[pl.ds(base, SUPER)], src_a)
    pltpu.sync_copy(dst_hbm.at[pl.ds(base, SUPER)], dst_a)
    plsc.subcore_barrier()

    ones16 = jnp.ones((16,), jnp.float32)

    pltpu.async_copy(table_hbm.at[src_a.at[0]], rows0_v, sem0)

    def process_super(u, src_c, dst_c, src_n, dst_n):
        @pl.when(u + 1 < SUPN)
        def _():
            off = base + (u + 1) * SUPER
            pltpu.async_copy(src_hbm.at[pl.ds(off, SUPER)], src_n, sem_is)
            pltpu.async_copy(dst_hbm.at[pl.ds(off, SUPER)], dst_n, sem_id)

        def dbody(k, carry):
            idx = dst_c[k // 8, pl.ds((k % 8) * 16, 16)]
            plsc.addupdate_scatter(deg_v, [idx >> 7, idx & 127], ones16)
            return carry

        lax.fori_loop(0, SUPER * 8, dbody, 0)

        def inner(i, carry):
            j0 = 2 * i
            pltpu.async_copy(table_hbm.at[src_c.at[j0 + 1]], rows1_v, sem1)
            pltpu.make_async_copy(table_hbm.at[src_c.at[j0]], rows0_v,
                                  sem0).wait()
            pltpu.sync_copy(rows0_v, acc_sh.at[dst_c.at[j0]], add=True)

            @pl.when(j0 + 2 < SUPER)
            def _():
                pltpu.async_copy(table_hbm.at[src_c.at[j0 + 2]], rows0_v,
                                 sem0)

            pltpu.make_async_copy(table_hbm.at[src_c.at[j0 + 1]], rows1_v,
                                  sem1).wait()
            pltpu.sync_copy(rows1_v, acc_sh.at[dst_c.at[j0 + 1]], add=True)
            return carry

        lax.fori_loop(0, SUPER // 2 - 1, inner, 0)

        jl = SUPER - 2
        pltpu.async_copy(table_hbm.at[src_c.at[jl + 1]], rows1_v, sem1)
        pltpu.make_async_copy(table_hbm.at[src_c.at[jl]], rows0_v,
                              sem0).wait()
        pltpu.sync_copy(rows0_v, acc_sh.at[dst_c.at[jl]], add=True)

        @pl.when(u + 1 < SUPN)
        def _():
            pltpu.make_async_copy(src_hbm.at[pl.ds(base, SUPER)], src_n,
                                  sem_is).wait()
            pltpu.make_async_copy(dst_hbm.at[pl.ds(base, SUPER)], dst_n,
                                  sem_id).wait()
            pltpu.async_copy(table_hbm.at[src_n.at[0]], rows0_v, sem0)

        pltpu.make_async_copy(table_hbm.at[src_c.at[jl + 1]], rows1_v,
                              sem1).wait()
        pltpu.sync_copy(rows1_v, acc_sh.at[dst_c.at[jl + 1]], add=True)

    def super_body(t, carry):
        u0 = 2 * t
        process_super(u0, src_a, dst_a, src_b, dst_b)
        process_super(u0 + 1, src_b, dst_b, src_a, dst_a)
        return carry

    lax.fori_loop(0, SUPN // 2, super_body, 0)
    plsc.subcore_barrier()
    pltpu.sync_copy(acc_sh.at[pl.ds(r0, ROWS_PER_TILE)],
                    feat_hbm.at[c, pl.ds(r0, ROWS_PER_TILE)])
    pltpu.sync_copy(deg_v, deg_hbm.at[pl.ds(w * (NPAD // D), NPAD // D)])


_RBLK = 1024


def _self_body(h_ref, ws_ref, b_ref, o_ref):
    o_ref[...] = (jnp.dot(h_ref[...], ws_ref[...],
                          preferred_element_type=jnp.float32) + b_ref[...])


def _self_mm(h, w_self, b):
    return pl.pallas_call(
        _self_body,
        grid=(NPAD // _RBLK,),
        in_specs=[
            pl.BlockSpec((_RBLK, D), lambda i: (i, 0)),
            pl.BlockSpec((D, D), lambda i: (0, 0)),
            pl.BlockSpec((1, D), lambda i: (0, 0)),
        ],
        out_specs=pl.BlockSpec((_RBLK, D), lambda i: (i, 0)),
        out_shape=jax.ShapeDtypeStruct((NPAD, D), jnp.float32),
    )(h, w_self, b)


def _combine_body(relu, s_ref, p_ref, degp_ref, wn_ref, o_ref):
    feat = p_ref[0] + p_ref[1]
    deg = lax.dot_general(degp_ref[...], jnp.ones((NW, 1), jnp.float32),
                          (((0,), (0,)), ((), ())),
                          preferred_element_type=jnp.float32)
    hn = feat / jnp.maximum(deg, 1.0)
    act = s_ref[...] + jnp.dot(hn, wn_ref[...],
                               preferred_element_type=jnp.float32)
    if relu:
        act = jnp.maximum(act, 0.0)
    o_ref[...] = act


def _combine(selfmm, p, degp, w_neigh, relu):
    return pl.pallas_call(
        functools.partial(_combine_body, relu),
        grid=(NPAD // _RBLK,),
        in_specs=[
            pl.BlockSpec((_RBLK, D), lambda i: (i, 0)),
            pl.BlockSpec((NC, _RBLK, D), lambda i: (0, i, 0)),
            pl.BlockSpec((NW, _RBLK), lambda i: (0, i)),
            pl.BlockSpec((D, D), lambda i: (0, 0)),
        ],
        out_specs=pl.BlockSpec((_RBLK, D), lambda i: (i, 0)),
        out_shape=jax.ShapeDtypeStruct((NPAD, D), jnp.float32),
    )(selfmm, p, degp, w_neigh)


_PAD = EPAD - E
_SRC_TMPL = np.zeros((EPAD,), np.int32)
_SRC_TMPL[E:] = np.arange(_PAD, dtype=np.int32) % N
_DST_TMPL = np.zeros((EPAD,), np.int32)
_DST_TMPL[E:] = N + np.arange(_PAD, dtype=np.int32) % (NPAD - N)


def _pad_edges(ei):
    src = lax.dynamic_update_slice(jnp.asarray(_SRC_TMPL),
                                   ei[0].astype(jnp.int32), (0,))
    dst = lax.dynamic_update_slice(jnp.asarray(_DST_TMPL),
                                   ei[1].astype(jnp.int32), (0,))
    return src.reshape(-1, CHUNK), dst.reshape(-1, CHUNK)


def kernel(x, edge_index0, edge_index1, W_self1, W_neigh1, b1,
           W_self2, W_neigh2, b2):
    src0, dst0 = _pad_edges(edge_index0)
    src1, dst1 = _pad_edges(edge_index1)
    zeros = jnp.zeros((NPAD, D), jnp.float32)
    xp = jnp.pad(x, ((0, NPAD - N), (0, 0)))
    b1r = b1.reshape(1, D)
    b2r = b2.reshape(1, D)

    p1, d1 = _sc_aggregate(x, src0, dst0, zeros)
    s1 = _self_mm(xp, W_self1, b1r)
    h = _combine(s1, p1, d1.reshape(NW, NPAD), W_neigh1, relu=True)
    p2, d2 = _sc_aggregate(h, src1, dst1, zeros)
    s2 = _self_mm(h, W_self2, b2r)
    out = _combine(s2, p2, d2.reshape(NW, NPAD), W_neigh2, relu=False)
    return out[:N]


# speedup vs baseline: 1.1048x; 1.0264x over previous
import functools

import jax
import jax.numpy as jnp
import numpy as np
from jax import lax
from jax.experimental import pallas as pl
from jax.experimental.pallas import tpu as pltpu
from jax.experimental.pallas import tpu_sc as plsc

N = 10000
NPAD = 10240
D = 128
E = 320000

NC = 2
NS = 16
NW = NC * NS
CHUNK = 128
NCHUNK = 80
EPW = NCHUNK * CHUNK
EPAD = NW * EPW
SUPER = 8
SUPN = NCHUNK // SUPER
ROWS_PER_TILE = NPAD // NS

_MESH = plsc.VectorSubcoreMesh(core_axis_name="c", subcore_axis_name="s")


@functools.partial(
    pl.kernel,
    out_type=(
        jax.ShapeDtypeStruct((NC, NPAD, D), jnp.float32),
        jax.ShapeDtypeStruct((NW * NPAD // D, D), jnp.float32),
    ),
    mesh=_MESH,
    compiler_params=pltpu.CompilerParams(use_tc_tiling_on_sc=True,
                                         needs_layout_passes=False),
    scratch_types=[
        pltpu.VMEM((SUPER, CHUNK), jnp.int32),
        pltpu.VMEM((SUPER, CHUNK), jnp.int32),
        pltpu.VMEM((SUPER, CHUNK), jnp.int32),
        pltpu.VMEM((SUPER, CHUNK), jnp.int32),
        pltpu.VMEM((CHUNK, D), jnp.float32),
        pltpu.VMEM((CHUNK, D), jnp.float32),
        pltpu.VMEM((NPAD // D, D), jnp.float32),
        pltpu.VMEM_SHARED((NPAD, D), jnp.float32),
        pltpu.SemaphoreType.DMA,
        pltpu.SemaphoreType.DMA,
        pltpu.SemaphoreType.DMA,
        pltpu.SemaphoreType.DMA,
    ],
)
def _sc_aggregate(table_hbm, src_hbm, dst_hbm, zeros_hbm,
                  feat_hbm, deg_hbm,
                  src_a, dst_a, src_b, dst_b, rows0_v, rows1_v, deg_v,
                  acc_sh, sem0, sem1, sem_is, sem_id):
    c = lax.axis_index("c")
    s = lax.axis_index("s")
    w = c * NS + s
    r0 = s * ROWS_PER_TILE
    base = w * NCHUNK
    pltpu.sync_copy(zeros_hbm.at[pl.ds(r0, ROWS_PER_TILE)],
                    acc_sh.at[pl.ds(r0, ROWS_PER_TILE)])
    pltpu.sync_copy(zeros_hbm.at[pl.ds(r0, NPAD // D)], deg_v)
    pltpu.sync_copy(src_hbm.at[pl.ds(base, SUPER)], src_a)
    pltpu.sync_copy(dst_hbm.at[pl.ds(base, SUPER)], dst_a)
    plsc.subcore_barrier()

    ones16 = jnp.ones((16,), jnp.float32)

    pltpu.async_copy(table_hbm.at[src_a.at[0]], rows0_v, sem0)

    def process_super(u, src_c, dst_c, src_n, dst_n):
        @pl.when(u + 1 < SUPN)
        def _():
            off = base + (u + 1) * SUPER
            pltpu.async_copy(src_hbm.at[pl.ds(off, SUPER)], src_n, sem_is)
            pltpu.async_copy(dst_hbm.at[pl.ds(off, SUPER)], dst_n, sem_id)

        def dbody(k, carry):
            idx = dst_c[k // 8, pl.ds((k % 8) * 16, 16)]
            plsc.addupdate_scatter(deg_v, [idx >> 7, idx & 127], ones16)
            return carry

        lax.fori_loop(0, SUPER * 8, dbody, 0)

        def inner(i, carry):
            j0 = 2 * i
            pltpu.async_copy(table_hbm.at[src_c.at[j0 + 1]], rows1_v, sem1)
            pltpu.make_async_copy(table_hbm.at[src_c.at[j0]], rows0_v,
                                  sem0).wait()
            pltpu.sync_copy(rows0_v, acc_sh.at[dst_c.at[j0]], add=True)

            @pl.when(j0 + 2 < SUPER)
            def _():
                pltpu.async_copy(table_hbm.at[src_c.at[j0 + 2]], rows0_v,
                                 sem0)

            pltpu.make_async_copy(table_hbm.at[src_c.at[j0 + 1]], rows1_v,
                                  sem1).wait()
            pltpu.sync_copy(rows1_v, acc_sh.at[dst_c.at[j0 + 1]], add=True)
            return carry

        lax.fori_loop(0, SUPER // 2 - 1, inner, 0)

        jl = SUPER - 2
        pltpu.async_copy(table_hbm.at[src_c.at[jl + 1]], rows1_v, sem1)
        pltpu.make_async_copy(table_hbm.at[src_c.at[jl]], rows0_v,
                              sem0).wait()
        pltpu.sync_copy(rows0_v, acc_sh.at[dst_c.at[jl]], add=True)

        @pl.when(u + 1 < SUPN)
        def _():
            pltpu.make_async_copy(src_hbm.at[pl.ds(base, SUPER)], src_n,
                                  sem_is).wait()
            pltpu.make_async_copy(dst_hbm.at[pl.ds(base, SUPER)], dst_n,
                                  sem_id).wait()
            pltpu.async_copy(table_hbm.at[src_n.at[0]], rows0_v, sem0)

        pltpu.make_async_copy(table_hbm.at[src_c.at[jl + 1]], rows1_v,
                              sem1).wait()
        pltpu.sync_copy(rows1_v, acc_sh.at[dst_c.at[jl + 1]], add=True)

    def super_body(t, carry):
        u0 = 2 * t
        process_super(u0, src_a, dst_a, src_b, dst_b)
        process_super(u0 + 1, src_b, dst_b, src_a, dst_a)
        return carry

    lax.fori_loop(0, SUPN // 2, super_body, 0)
    plsc.subcore_barrier()
    pltpu.sync_copy(acc_sh.at[pl.ds(r0, ROWS_PER_TILE)],
                    feat_hbm.at[c, pl.ds(r0, ROWS_PER_TILE)])
    pltpu.sync_copy(deg_v, deg_hbm.at[pl.ds(w * (NPAD // D), NPAD // D)])


_RBLK = 1024


def _self_body(h_ref, ws_ref, b_ref, o_ref):
    o_ref[...] = (jnp.dot(h_ref[...], ws_ref[...],
                          preferred_element_type=jnp.float32) + b_ref[...])


def _self_mm(h, w_self, b):
    return pl.pallas_call(
        _self_body,
        grid=(NPAD // _RBLK,),
        in_specs=[
            pl.BlockSpec((_RBLK, D), lambda i: (i, 0)),
            pl.BlockSpec((D, D), lambda i: (0, 0)),
            pl.BlockSpec((1, D), lambda i: (0, 0)),
        ],
        out_specs=pl.BlockSpec((_RBLK, D), lambda i: (i, 0)),
        out_shape=jax.ShapeDtypeStruct((NPAD, D), jnp.float32),
    )(h, w_self, b)


def _combine_body(relu, s_ref, p_ref, degp_ref, wn_ref, o_ref):
    feat = p_ref[0] + p_ref[1]
    deg = lax.dot_general(degp_ref[...], jnp.ones((NW, 1), jnp.float32),
                          (((0,), (0,)), ((), ())),
                          preferred_element_type=jnp.float32)
    hn = feat / jnp.maximum(deg, 1.0)
    act = s_ref[...] + jnp.dot(hn, wn_ref[...],
                               preferred_element_type=jnp.float32)
    if relu:
        act = jnp.maximum(act, 0.0)
    o_ref[...] = act


def _combine(selfmm, p, degp, w_neigh, relu):
    return pl.pallas_call(
        functools.partial(_combine_body, relu),
        grid=(NPAD // _RBLK,),
        in_specs=[
            pl.BlockSpec((_RBLK, D), lambda i: (i, 0)),
            pl.BlockSpec((NC, _RBLK, D), lambda i: (0, i, 0)),
            pl.BlockSpec((NW, _RBLK), lambda i: (0, i)),
            pl.BlockSpec((D, D), lambda i: (0, 0)),
        ],
        out_specs=pl.BlockSpec((_RBLK, D), lambda i: (i, 0)),
        out_shape=jax.ShapeDtypeStruct((NPAD, D), jnp.float32),
    )(selfmm, p, degp, w_neigh)


_PAD = EPAD - E
_SRC_TMPL = np.zeros((EPAD,), np.int32)
_SRC_TMPL[E:] = np.arange(_PAD, dtype=np.int32) % N
_DST_TMPL = np.zeros((EPAD,), np.int32)
_DST_TMPL[E:] = N + np.arange(_PAD, dtype=np.int32) % (NPAD - N)


def _pad_edges(ei):
    src = jnp.concatenate([ei[0].astype(jnp.int32),
                           jnp.asarray(_SRC_TMPL[E:])])
    dst = jnp.concatenate([ei[1].astype(jnp.int32),
                           jnp.asarray(_DST_TMPL[E:])])
    return src.reshape(-1, CHUNK), dst.reshape(-1, CHUNK)


def kernel(x, edge_index0, edge_index1, W_self1, W_neigh1, b1,
           W_self2, W_neigh2, b2):
    src0, dst0 = _pad_edges(edge_index0)
    src1, dst1 = _pad_edges(edge_index1)
    zeros = jnp.zeros((NPAD, D), jnp.float32)
    xp = jnp.pad(x, ((0, NPAD - N), (0, 0)))
    b1r = b1.reshape(1, D)
    b2r = b2.reshape(1, D)

    p1, d1 = _sc_aggregate(x, src0, dst0, zeros)
    s1 = _self_mm(xp, W_self1, b1r)
    h = _combine(s1, p1, d1.reshape(NW, NPAD), W_neigh1, relu=True)
    p2, d2 = _sc_aggregate(h, src1, dst1, zeros)
    s2 = _self_mm(h, W_self2, b2r)
    out = _combine(s2, p2, d2.reshape(NW, NPAD), W_neigh2, relu=False)
    return out[:N]
